# baseline (speedup 1.0000x reference)
import jax
import jax.numpy as jnp
from jax import lax
from jax.experimental import pallas as pl
from jax.experimental.pallas import tpu as pltpu
from jax.experimental.pallas import tpu_sc as plsc

VOCAB = 1_000_000
D_MODEL = 64
BATCH = 16384
SEQ = 50

NC = 2
NS = 16
NW = NC * NS

NBLK_A = VOCAB // 128
BPW_A = (NBLK_A + NW - 1) // NW
SCR_ROWS = VOCAB // 2
TAIL_V0 = NBLK_A * 128

BPW_B = BATCH // NW


def _iota16():
    return lax.broadcasted_iota(jnp.int32, (16,), 0)


def _gather_body(idsT_hbm, scr_hbm, out_hbm, stage_v, qh_v, in2_v, out3_v,
                 gsem, wsem):
    wid = lax.axis_index("s") * NC + lax.axis_index("c")
    b0w = pl.multiple_of(wid * BPW_B, BPW_B)

    def gather(c, slot):
        pltpu.async_copy(
            scr_hbm.at[qh_v.at[0, pl.ds(c * 128, 128)]],
            in2_v.at[slot, :, pl.ds(0, 128)],
            gsem.at[slot],
        )

    def gather_wait(slot):
        pltpu.make_async_copy(
            scr_hbm.at[qh_v.at[0, pl.ds(0, 128)]],
            in2_v.at[slot, :, pl.ds(0, 128)],
            gsem.at[slot],
        ).wait()

    def write(s, c, slot):
        bc = pl.multiple_of(b0w + c * 128, 128)
        pltpu.async_copy(
            out3_v.at[slot], out_hbm.at[s, :, pl.ds(bc, 128)],
            wsem.at[slot],
        )

    def write_wait(slot):
        pltpu.make_async_copy(
            out3_v.at[slot], out_hbm.at[0, :, pl.ds(0, 128)], wsem.at[slot]
        ).wait()

    bvecs = [_iota16() + 16 * mm for mm in range(8)]

    def transpose_chunk(c, slot):
        for m in range(8):
            pv = qh_v[1, pl.ds(c * 128 + 16 * m, 16)]

            @plsc.parallel_loop(0, 64, unroll=8)
            def _d(d):
                val = plsc.load_gather(in2_v.at[slot], [bvecs[m], pv + d])
                out3_v[slot, d, pl.ds(16 * m, 16)] = val

    def do_s(s, si):
        @plsc.parallel_loop(0, 32, unroll=4)
        def _m2(m2):
            v16 = stage_v[si, pl.ds(16 * m2, 16)]
            qh_v[0, pl.ds(16 * m2, 16)] = lax.shift_right_logical(v16, 1)
            qh_v[1, pl.ds(16 * m2, 16)] = lax.shift_left(
                lax.bitwise_and(v16, 1), 6)

        gather(0, 0)
        gather(1, 1)
        for c in range(4):
            slot = c % 2
            gather_wait(slot)
            use_idx = s * 4 + c

            @pl.when(use_idx >= 2)
            def _():
                write_wait(slot)

            transpose_chunk(c, slot)
            if c + 2 < 4:
                gather(c + 2, slot)
            write(s, c, slot)

    @pl.loop(0, 6)
    def _sg(sg):
        s0 = pl.multiple_of(sg * 8, 8)
        pltpu.sync_copy(
            idsT_hbm.at[pl.ds(s0, 8), pl.ds(b0w, BPW_B)], stage_v
        )

        @pl.loop(0, 8)
        def _si(si):
            do_s(s0 + si, si)

    pltpu.sync_copy(
        idsT_hbm.at[pl.ds(48, 2), pl.ds(b0w, BPW_B)], stage_v.at[pl.ds(0, 2)]
    )

    @pl.loop(0, 2)
    def _se(si):
        do_s(48 + si, si)

    write_wait(0)
    write_wait(1)


def _make_calls():
    mesh = plsc.VectorSubcoreMesh(
        core_axis_name="c", subcore_axis_name="s", num_cores=NC,
        num_subcores=NS,
    )
    params = pltpu.CompilerParams(
        use_tc_tiling_on_sc=True, needs_layout_passes=False
    )
    gather_call = pl.kernel(
        _gather_body,
        out_type=jax.ShapeDtypeStruct((SEQ, D_MODEL, BATCH), jnp.float32),
        mesh=mesh,
        scratch_types=[
            pltpu.VMEM((8, BPW_B), jnp.int32),
            pltpu.VMEM((2, BPW_B), jnp.int32),
            pltpu.VMEM((2, 128, 129), jnp.float32),
            pltpu.VMEM((2, 64, 128), jnp.float32),
            pltpu.SemaphoreType.DMA((2,)),
            pltpu.SemaphoreType.DMA((2,)),
        ],
        compiler_params=params,
    )
    return gather_call


def kernel(input_ids, delta_table):
    gather_call = _make_calls()
    scr = jnp.reshape(delta_table, (SCR_ROWS, 128))
    idsT = jnp.transpose(input_ids.astype(jnp.int32))
    outT = gather_call(idsT, scr)
    return jnp.transpose(outT, (2, 0, 1))


# speedup vs baseline: 1.0577x; 1.0577x over previous
import jax
import jax.numpy as jnp
from jax import lax
from jax.experimental import pallas as pl
from jax.experimental.pallas import tpu as pltpu
from jax.experimental.pallas import tpu_sc as plsc

VOCAB = 1_000_000
D_MODEL = 64
BATCH = 16384
SEQ = 50

NC = 2
NS = 16
NW = NC * NS

NBLK_A = VOCAB // 128
BPW_A = (NBLK_A + NW - 1) // NW
SCR_ROWS = VOCAB // 2
TAIL_V0 = NBLK_A * 128

BPW_B = BATCH // NW


def _iota16():
    return lax.broadcasted_iota(jnp.int32, (16,), 0)


def _gather_body(idsT_hbm, scr_hbm, out_hbm, stage_v, qh_v, in2_v, out3_v,
                 gsem, wsem):
    wid = lax.axis_index("s") * NC + lax.axis_index("c")
    b0w = pl.multiple_of(wid * BPW_B, BPW_B)

    def gather(c, slot):
        pltpu.async_copy(
            scr_hbm.at[qh_v.at[0, pl.ds(c * 128, 128)]],
            in2_v.at[slot],
            gsem.at[slot],
        )

    def gather_wait(slot):
        pltpu.make_async_copy(
            scr_hbm.at[qh_v.at[0, pl.ds(0, 128)]],
            in2_v.at[slot],
            gsem.at[slot],
        ).wait()

    def write(s, c, slot):
        bc = pl.multiple_of(b0w + c * 128, 128)
        pltpu.async_copy(
            out3_v.at[slot], out_hbm.at[s, :, pl.ds(bc, 128)],
            wsem.at[slot],
        )

    def write_wait(slot):
        pltpu.make_async_copy(
            out3_v.at[slot], out_hbm.at[0, :, pl.ds(0, 128)], wsem.at[slot]
        ).wait()

    bvecs = [_iota16() + 16 * mm for mm in range(8)]

    def transpose_chunk(c, slot):
        for m in range(8):
            pv = qh_v[1, pl.ds(c * 128 + 16 * m, 16)]

            @plsc.parallel_loop(0, 64, unroll=16)
            def _d(d):
                val = plsc.load_gather(in2_v.at[slot], [bvecs[m], pv + d])
                out3_v[slot, d, pl.ds(16 * m, 16)] = val

    def do_s(s, si):
        @plsc.parallel_loop(0, 32, unroll=8)
        def _m2(m2):
            v16 = stage_v[si, pl.ds(16 * m2, 16)]
            qh_v[0, pl.ds(16 * m2, 16)] = lax.shift_right_logical(v16, 1)
            qh_v[1, pl.ds(16 * m2, 16)] = lax.shift_left(
                lax.bitwise_and(v16, 1), 6)

        gather(0, 0)
        gather(1, 1)
        for c in range(4):
            slot = c % 2
            gather_wait(slot)
            use_idx = s * 4 + c

            @pl.when(use_idx >= 2)
            def _():
                write_wait(slot)

            transpose_chunk(c, slot)
            if c + 2 < 4:
                gather(c + 2, slot)
            write(s, c, slot)

    @pl.loop(0, 6)
    def _sg(sg):
        s0 = pl.multiple_of(sg * 8, 8)
        pltpu.sync_copy(
            idsT_hbm.at[pl.ds(s0, 8), pl.ds(b0w, BPW_B)], stage_v
        )

        @pl.loop(0, 8)
        def _si(si):
            do_s(s0 + si, si)

    pltpu.sync_copy(
        idsT_hbm.at[pl.ds(48, 2), pl.ds(b0w, BPW_B)], stage_v.at[pl.ds(0, 2)]
    )

    @pl.loop(0, 2)
    def _se(si):
        do_s(48 + si, si)

    write_wait(0)
    write_wait(1)


def _make_calls():
    mesh = plsc.VectorSubcoreMesh(
        core_axis_name="c", subcore_axis_name="s", num_cores=NC,
        num_subcores=NS,
    )
    params = pltpu.CompilerParams(
        use_tc_tiling_on_sc=True, needs_layout_passes=False
    )
    gather_call = pl.kernel(
        _gather_body,
        out_type=jax.ShapeDtypeStruct((SEQ, D_MODEL, BATCH), jnp.float32),
        mesh=mesh,
        scratch_types=[
            pltpu.VMEM((8, BPW_B), jnp.int32),
            pltpu.VMEM((2, BPW_B), jnp.int32),
            pltpu.VMEM((2, 128, 128), jnp.float32),
            pltpu.VMEM((2, 64, 128), jnp.float32),
            pltpu.SemaphoreType.DMA((2,)),
            pltpu.SemaphoreType.DMA((2,)),
        ],
        compiler_params=params,
    )
    return gather_call


def kernel(input_ids, delta_table):
    gather_call = _make_calls()
    scr = jnp.reshape(delta_table, (SCR_ROWS, 128))
    idsT = jnp.transpose(input_ids.astype(jnp.int32))
    outT = gather_call(idsT, scr)
    return jnp.transpose(outT, (2, 0, 1))


# speedup vs baseline: 1.1558x; 1.0927x over previous
import functools

import jax
import jax.numpy as jnp
from jax import lax
from jax.experimental import pallas as pl
from jax.experimental.pallas import tpu as pltpu
from jax.experimental.pallas import tpu_sc as plsc

VOCAB = 1_000_000
D_MODEL = 64
BATCH = 16384
SEQ = 50

NC = 2
NS = 16
NW = NC * NS
N_IDX = BATCH * SEQ
CHUNK = 128
N_CHUNKS = N_IDX // CHUNK
CPW = N_CHUNKS // NW
SLOTS = 8
NGROUPS = CPW // SLOTS


def _sc_body(ids_hbm, table_hbm, out_hbm, idx_v, rows_v, gsem, wsem):
    wid = lax.axis_index("s") * NC + lax.axis_index("c")
    chunk0 = wid * CPW

    pltpu.sync_copy(ids_hbm.at[pl.ds(chunk0, CPW)], idx_v)

    def gather(j_local, b):
        pltpu.async_copy(
            table_hbm.at[idx_v.at[j_local]], rows_v.at[b], gsem.at[b]
        )

    def gather_wait(b):
        pltpu.make_async_copy(
            table_hbm.at[idx_v.at[0]], rows_v.at[b], gsem.at[b]
        ).wait()

    def write(j_local, b):
        off = (chunk0 + j_local) * CHUNK
        pltpu.async_copy(rows_v.at[b], out_hbm.at[pl.ds(off, CHUNK)], wsem.at[b])

    def write_wait(b):
        pltpu.make_async_copy(
            rows_v.at[b], out_hbm.at[pl.ds(0, CHUNK)], wsem.at[b]
        ).wait()

    for b in range(SLOTS):
        gather(b, b)

    @pl.loop(0, NGROUPS)
    def _grp(g):
        base = g * SLOTS
        for b in range(SLOTS):
            gather_wait(b)
            write(base + b, b)
        for b in range(SLOTS):
            write_wait(b)

            @pl.when(g < NGROUPS - 1)
            def _():
                gather(base + SLOTS + b, b)


def _sc_lookup(ids2d, table):
    mesh = plsc.VectorSubcoreMesh(
        core_axis_name="c", subcore_axis_name="s", num_cores=NC, num_subcores=NS
    )
    fn = pl.kernel(
        _sc_body,
        out_type=jax.ShapeDtypeStruct((N_IDX, D_MODEL), jnp.float32),
        mesh=mesh,
        scratch_types=[
            pltpu.VMEM((CPW, CHUNK), jnp.int32),
            pltpu.VMEM((SLOTS, CHUNK, D_MODEL), jnp.float32),
            pltpu.SemaphoreType.DMA((SLOTS,)),
            pltpu.SemaphoreType.DMA((SLOTS,)),
        ],
        compiler_params=pltpu.CompilerParams(use_tc_tiling_on_sc=False),
    )
    return fn(ids2d, table)


def kernel(input_ids, delta_table):
    ids2d = jnp.reshape(input_ids.astype(jnp.int32), (N_CHUNKS, CHUNK))
    out = _sc_lookup(ids2d, delta_table)
    return jnp.reshape(out, (BATCH, SEQ, D_MODEL))
